# Initial kernel scaffold; baseline (speedup 1.0000x reference)
import jax
import jax.numpy as jnp
from jax.experimental import pallas as pl


def kernel(x, edge_index, edge_weight, W1, b1, gamma1, beta1, W2, b2, gamma2, beta2, Wih1, Whh1, bih1, bhh1, Wih2, Whh2, bih2, bhh2, Wlin, blin):
    raise NotImplementedError("write your pallas kernel here")


# speedup vs baseline: 8.8026x; 8.8026x over previous
import functools

import jax
import jax.numpy as jnp
from jax import lax
from jax.experimental import pallas as pl
from jax.experimental.pallas import tpu as pltpu
from jax.experimental.pallas import tpu_sc as plsc

N = 10000
E = 320000
D = 128

NC = 2
NS = 16
NW = NC * NS
EPW = E // NW
CK = 80
NCHUNK = EPW // CK
NP = 10240
RPT = NP // NS


def _tile_copy(src, dst, s):
    pltpu.sync_copy(src.at[pl.ds(s * RPT, RPT)], dst.at[pl.ds(s * RPT, RPT)])

@functools.cache
def _mesh():
    return plsc.VectorSubcoreMesh(core_axis_name="c", subcore_axis_name="s",
                                  num_cores=NC, num_subcores=NS)


def _sc_deg_body(col_hbm, ew_hbm, zeros1_hbm, out_hbm, col_v, ew_v, acc):
    c = lax.axis_index("c")
    s = lax.axis_index("s")
    wid = s * NC + c
    _tile_copy(zeros1_hbm, acc, s)
    plsc.subcore_barrier()

    def chunk(ci, carry):
        base = wid * EPW + ci * CK
        pltpu.sync_copy(col_hbm.at[pl.ds(base, CK)], col_v)
        pltpu.sync_copy(ew_hbm.at[pl.ds(base, CK)], ew_v)
        pltpu.sync_copy(ew_v, acc.at[col_v], add=True)
        return carry

    lax.fori_loop(0, NCHUNK, chunk, 0)
    plsc.subcore_barrier()
    _tile_copy(acc, out_hbm.at[c], s)


@functools.cache
def _sc_deg():
    return pl.kernel(
        _sc_deg_body,
        out_type=jax.ShapeDtypeStruct((NC, NP), jnp.float32),
        mesh=_mesh(),
        scratch_types=[
            pltpu.VMEM((CK,), jnp.int32),
            pltpu.VMEM((CK,), jnp.float32),
            pltpu.VMEM_SHARED((NP,), jnp.float32),
        ],
    )


def _sc_agg_body(row_hbm, col_hbm, ew_hbm, xws_hbm, zeros2_hbm, out_hbm,
                 row_v, col_v, ew_v, rows_v, acc, sem):
    c = lax.axis_index("c")
    s = lax.axis_index("s")
    wid = s * NC + c
    _tile_copy(zeros2_hbm, acc, s)
    plsc.subcore_barrier()

    def chunk(ci, carry):
        base = wid * EPW + ci * CK
        pltpu.sync_copy(row_hbm.at[pl.ds(base, CK)], row_v)
        pltpu.sync_copy(col_hbm.at[pl.ds(base, CK)], col_v)
        pltpu.sync_copy(ew_hbm.at[pl.ds(base, CK)], ew_v)
        pltpu.async_copy(xws_hbm.at[row_v], rows_v, sem).wait()

        def group(g, gcarry):
            w16 = ew_v[pl.ds(g * 16, 16)]
            for l in range(16):
                e = g * 16 + l
                w = w16[l]
                for j in range(D // 16):
                    sl = pl.ds(j * 16, 16)
                    rows_v[e, sl] = rows_v[e, sl] * w
            return gcarry

        lax.fori_loop(0, CK // 16, group, 0)
        pltpu.sync_copy(rows_v, acc.at[col_v], add=True)
        return carry

    lax.fori_loop(0, NCHUNK, chunk, 0)
    plsc.subcore_barrier()
    _tile_copy(acc, out_hbm.at[c], s)


@functools.cache
def _sc_agg():
    return pl.kernel(
        _sc_agg_body,
        out_type=jax.ShapeDtypeStruct((NC, NP, D), jnp.float32),
        mesh=_mesh(),
        scratch_types=[
            pltpu.VMEM((CK,), jnp.int32),
            pltpu.VMEM((CK,), jnp.int32),
            pltpu.VMEM((CK,), jnp.float32),
            pltpu.VMEM((CK, D), jnp.float32),
            pltpu.VMEM_SHARED((NP, D), jnp.float32),
            pltpu.SemaphoreType.DMA,
        ],
    )


def _tc_a_body(degp_ref, x_ref, w1_ref, xws_ref, dis_ref):
    deg = degp_ref[0, :N] + degp_ref[1, :N] + 1.0
    dis = lax.rsqrt(deg)
    xw = lax.dot_general(x_ref[...], w1_ref[...],
                         (((1,), (1,)), ((), ())),
                         preferred_element_type=jnp.float32)
    xws_ref[...] = xw * dis
    dis_ref[...] = dis


def _tc_a(degp, x, W1):
    return pl.pallas_call(
        _tc_a_body,
        out_shape=(jax.ShapeDtypeStruct((N, D), jnp.float32),
                   jax.ShapeDtypeStruct((N, 1), jnp.float32)),
    )(degp, x, W1)


def _bn_cols(a, gamma, beta):
    mu = jnp.mean(a, axis=0, keepdims=True)
    d = a - mu
    var = jnp.mean(d * d, axis=0, keepdims=True)
    return d * lax.rsqrt(var + 1e-5) * gamma + beta


def _tc_b_body(aggp_ref, xws_ref, dis_ref, b1_ref, g1_ref, be1_ref, w2_ref,
               h_ref, xws2_ref):
    dis = dis_ref[...]
    a = dis * (aggp_ref[0, :N] + aggp_ref[1, :N] + xws_ref[...]) + b1_ref[...]
    a = jnp.maximum(a, 0.0)
    h = _bn_cols(a, g1_ref[...], be1_ref[...])
    h_ref[...] = h
    xw2 = lax.dot_general(h, w2_ref[...], (((1,), (1,)), ((), ())),
                          preferred_element_type=jnp.float32)
    xws2_ref[...] = xw2 * dis


def _tc_b(aggp, xws1, dis, b1, gamma1, beta1, W2):
    return pl.pallas_call(
        _tc_b_body,
        out_shape=(jax.ShapeDtypeStruct((N, D), jnp.float32),
                   jax.ShapeDtypeStruct((N, D), jnp.float32)),
    )(aggp, xws1, dis, b1.reshape(1, D), gamma1.reshape(1, D),
      beta1.reshape(1, D), W2)


def _tc_c_body(aggp_ref, xws_ref, dis_ref, b2_ref, g2_ref, be2_ref,
               h_ref, x_ref, wih1_ref, bsum1_ref, wih2_ref, bsum2_ref,
               wlin_ref, blin_ref, out_ref):
    dis = dis_ref[...]
    a = dis * (aggp_ref[0, :N] + aggp_ref[1, :N] + xws_ref[...]) + b2_ref[...]
    a = jnp.maximum(a, 0.0)
    h2 = _bn_cols(a, g2_ref[...], be2_ref[...])
    h = h_ref[...]

    wih1 = wih1_ref[...]
    g1 = (lax.dot_general(h, wih1[:, :D], (((1,), (1,)), ((), ())),
                          preferred_element_type=jnp.float32)
          + lax.dot_general(h2, wih1[:, D:], (((1,), (1,)), ((), ())),
                            preferred_element_type=jnp.float32)
          + bsum1_ref[...])
    i1 = jax.nn.sigmoid(g1[:, :D])
    gg1 = jnp.tanh(g1[:, 2 * D:3 * D])
    o1 = jax.nn.sigmoid(g1[:, 3 * D:])
    H1 = o1 * jnp.tanh(i1 * gg1)

    g2 = (lax.dot_general(H1, wih2_ref[...], (((1,), (1,)), ((), ())),
                          preferred_element_type=jnp.float32)
          + bsum2_ref[...])
    i2 = jax.nn.sigmoid(g2[:, :D])
    gg2 = jnp.tanh(g2[:, 2 * D:3 * D])
    o2 = jax.nn.sigmoid(g2[:, 3 * D:])
    H2 = o2 * jnp.tanh(i2 * gg2)

    wlin = wlin_ref[...]
    Hr1 = jnp.maximum(H1, 0.0)
    Hr2 = jnp.maximum(H2, 0.0)
    Hr3 = jnp.maximum(x_ref[...], 0.0)
    o = (lax.dot_general(Hr1, wlin[:, :D], (((1,), (1,)), ((), ())),
                         preferred_element_type=jnp.float32)
         + lax.dot_general(Hr2, wlin[:, D:2 * D], (((1,), (1,)), ((), ())),
                           preferred_element_type=jnp.float32)
         + lax.dot_general(Hr3, wlin[:, 2 * D:], (((1,), (1,)), ((), ())),
                           preferred_element_type=jnp.float32)
         + blin_ref[...])
    out_ref[...] = jnp.tanh(o)


def _tc_c(aggp, xws2, dis, b2, gamma2, beta2, h, x,
          Wih1, bsum1, Wih2, bsum2, Wlin, blin):
    return pl.pallas_call(
        _tc_c_body,
        out_shape=jax.ShapeDtypeStruct((N, 1), jnp.float32),
    )(aggp, xws2, dis, b2.reshape(1, D), gamma2.reshape(1, D),
      beta2.reshape(1, D), h, x, Wih1, bsum1.reshape(1, 4 * D),
      Wih2, bsum2.reshape(1, 4 * D), Wlin, blin.reshape(1, 1))


def kernel(x, edge_index, edge_weight, W1, b1, gamma1, beta1,
           W2, b2, gamma2, beta2, Wih1, Whh1, bih1, bhh1,
           Wih2, Whh2, bih2, bhh2, Wlin, blin):
    row = edge_index[0]
    col = edge_index[1]
    zeros1 = jnp.zeros((NP,), jnp.float32)
    zeros2 = jnp.zeros((NP, D), jnp.float32)

    degp = _sc_deg()(col, edge_weight, zeros1)
    xws1, dis = _tc_a(degp.reshape(NC, NP, 1), x, W1)
    aggp1 = _sc_agg()(row, col, edge_weight, xws1, zeros2)
    h, xws2 = _tc_b(aggp1, xws1, dis, b1, gamma1, beta1, W2)
    aggp2 = _sc_agg()(row, col, edge_weight, xws2, zeros2)
    out = _tc_c(aggp2, xws2, dis, b2, gamma2, beta2, h, x,
                Wih1, bih1 + bhh1, Wih2, bih2 + bhh2, Wlin, blin)
    return out


# speedup vs baseline: 24.3769x; 2.7693x over previous
import functools

import jax
import jax.numpy as jnp
from jax import lax
from jax.experimental import pallas as pl
from jax.experimental.pallas import tpu as pltpu
from jax.experimental.pallas import tpu_sc as plsc

N = 10000
E = 320000
D = 128

NC = 2
NS = 16
NW = NC * NS
CK = 128
NCH = 80
PADC = NW * NCH
E2 = PADC * CK
NBUF = 4
NP = 10240
RPT = NP // NS


def _tile_copy(src, dst, s):
    pltpu.sync_copy(src.at[pl.ds(s * RPT, RPT)], dst.at[pl.ds(s * RPT, RPT)])

@functools.cache
def _mesh():
    return plsc.VectorSubcoreMesh(core_axis_name="c", subcore_axis_name="s",
                                  num_cores=NC, num_subcores=NS)


def _sc_deg_body(col_hbm, ew_hbm, zeros1_hbm, out_hbm, col_all, ew_all, acc,
                 sem):
    c = lax.axis_index("c")
    s = lax.axis_index("s")
    wid = s * NC + c
    pltpu.sync_copy(col_hbm.at[pl.ds(wid * NCH, NCH)], col_all)
    pltpu.sync_copy(ew_hbm.at[pl.ds(wid * NCH, NCH)], ew_all)
    _tile_copy(zeros1_hbm, acc, s)
    plsc.subcore_barrier()

    def octet(p, carry):
        for k in range(8):
            i = p * 8 + k
            pltpu.async_copy(ew_all.at[i], acc.at[col_all.at[i]], sem,
                             add=True)
        for k in range(8):
            i = p * 8 + k
            pltpu.make_async_copy(ew_all.at[i], acc.at[col_all.at[i]],
                                  sem).wait()
        return carry

    lax.fori_loop(0, NCH // 8, octet, 0)
    plsc.subcore_barrier()
    _tile_copy(acc, out_hbm.at[c], s)


@functools.cache
def _sc_deg():
    return pl.kernel(
        _sc_deg_body,
        out_type=jax.ShapeDtypeStruct((NC, NP), jnp.float32),
        mesh=_mesh(),
        scratch_types=[
            pltpu.VMEM((NCH, CK), jnp.int32),
            pltpu.VMEM((NCH, CK), jnp.float32),
            pltpu.VMEM_SHARED((NP,), jnp.float32),
            pltpu.SemaphoreType.DMA,
        ],
    )


def _sc_agg_body(row_hbm, col_hbm, ew_hbm, xws_hbm, zeros2_hbm, out_hbm,
                 col_all, row_v, ew_v, rows_v, acc, sem_i, sem_g, sem_s):
    c = lax.axis_index("c")
    s = lax.axis_index("s")
    wid = s * NC + c
    base = wid * NCH
    pltpu.sync_copy(col_hbm.at[pl.ds(base, NCH)], col_all)
    _tile_copy(zeros2_hbm, acc, s)

    def idx_start(i, b):
        pltpu.async_copy(row_hbm.at[base + i], row_v.at[b], sem_i.at[b])
        pltpu.async_copy(ew_hbm.at[base + i], ew_v.at[b], sem_i.at[b])

    def idx_wait(i, b):
        pltpu.make_async_copy(row_hbm.at[base + i], row_v.at[b],
                              sem_i.at[b]).wait()
        pltpu.make_async_copy(ew_hbm.at[base + i], ew_v.at[b],
                              sem_i.at[b]).wait()

    def gather_start(i, b):
        pltpu.async_copy(xws_hbm.at[row_v.at[b]], rows_v.at[b], sem_g.at[b])

    def gather_wait(i, b):
        pltpu.make_async_copy(xws_hbm.at[row_v.at[b]], rows_v.at[b],
                              sem_g.at[b]).wait()

    def scatter_start(i, b):
        pltpu.async_copy(rows_v.at[b], acc.at[col_all.at[i]], sem_s.at[b],
                         add=True)

    def scatter_wait(i, b):
        pltpu.make_async_copy(rows_v.at[b], acc.at[col_all.at[i]],
                              sem_s.at[b]).wait()

    idx_start(0, 0)
    idx_start(1, 1)
    plsc.subcore_barrier()
    idx_wait(0, 0)
    gather_start(0, 0)

    def pair(p, carry):
        for k in range(2):
            i = p * 2 + k
            b = k
            nb = 1 - k

            @pl.when(i + 1 < NCH)
            def _():
                idx_wait(i + 1, nb)

                @pl.when(i >= 1)
                def _():
                    scatter_wait(i - 1, nb)
                gather_start(i + 1, nb)

            gather_wait(i, b)

            def group(g, gcarry):
                w16 = ew_v[b, pl.ds(g * 16, 16)]
                for l in range(16):
                    e = g * 16 + l
                    w = w16[l]
                    for j in range(D // 16):
                        sl = pl.ds(j * 16, 16)
                        rows_v[b, e, sl] = rows_v[b, e, sl] * w
                return gcarry

            lax.fori_loop(0, CK // 16, group, 0)
            scatter_start(i, b)

            @pl.when(i + 2 < NCH)
            def _():
                idx_start(i + 2, b)
        return carry

    lax.fori_loop(0, NCH // 2, pair, 0)
    scatter_wait(NCH - 2, 0)
    scatter_wait(NCH - 1, 1)
    plsc.subcore_barrier()
    _tile_copy(acc, out_hbm.at[c], s)


@functools.cache
def _sc_agg():
    return pl.kernel(
        _sc_agg_body,
        out_type=jax.ShapeDtypeStruct((NC, NP, D), jnp.float32),
        mesh=_mesh(),
        scratch_types=[
            pltpu.VMEM((NCH, CK), jnp.int32),
            pltpu.VMEM((2, CK), jnp.int32),
            pltpu.VMEM((2, CK), jnp.float32),
            pltpu.VMEM((2, CK, D), jnp.float32),
            pltpu.VMEM_SHARED((NP, D), jnp.float32),
            pltpu.SemaphoreType.DMA((2,)),
            pltpu.SemaphoreType.DMA((2,)),
            pltpu.SemaphoreType.DMA((2,)),
        ],
    )


def _tc_a_body(degp_ref, x_ref, w1_ref, xws_ref, dis_ref):
    deg = degp_ref[0, :N] + degp_ref[1, :N] + 1.0
    dis = lax.rsqrt(deg)
    xw = lax.dot_general(x_ref[...], w1_ref[...],
                         (((1,), (1,)), ((), ())),
                         preferred_element_type=jnp.float32)
    xws_ref[...] = xw * dis
    dis_ref[...] = dis


def _tc_a(degp, x, W1):
    return pl.pallas_call(
        _tc_a_body,
        out_shape=(jax.ShapeDtypeStruct((N, D), jnp.float32),
                   jax.ShapeDtypeStruct((N, 1), jnp.float32)),
    )(degp, x, W1)


def _bn_cols(a, gamma, beta):
    mu = jnp.mean(a, axis=0, keepdims=True)
    d = a - mu
    var = jnp.mean(d * d, axis=0, keepdims=True)
    return d * lax.rsqrt(var + 1e-5) * gamma + beta


def _tc_b_body(aggp_ref, xws_ref, dis_ref, b1_ref, g1_ref, be1_ref, w2_ref,
               h_ref, xws2_ref):
    dis = dis_ref[...]
    a = dis * (aggp_ref[0, :N] + aggp_ref[1, :N] + xws_ref[...]) + b1_ref[...]
    a = jnp.maximum(a, 0.0)
    h = _bn_cols(a, g1_ref[...], be1_ref[...])
    h_ref[...] = h
    xw2 = lax.dot_general(h, w2_ref[...], (((1,), (1,)), ((), ())),
                          preferred_element_type=jnp.float32)
    xws2_ref[...] = xw2 * dis


def _tc_b(aggp, xws1, dis, b1, gamma1, beta1, W2):
    return pl.pallas_call(
        _tc_b_body,
        out_shape=(jax.ShapeDtypeStruct((N, D), jnp.float32),
                   jax.ShapeDtypeStruct((N, D), jnp.float32)),
    )(aggp, xws1, dis, b1.reshape(1, D), gamma1.reshape(1, D),
      beta1.reshape(1, D), W2)


def _tc_c_body(aggp_ref, xws_ref, dis_ref, b2_ref, g2_ref, be2_ref,
               h_ref, x_ref, wih1_ref, bsum1_ref, wih2_ref, bsum2_ref,
               wlin_ref, blin_ref, out_ref):
    dis = dis_ref[...]
    a = dis * (aggp_ref[0, :N] + aggp_ref[1, :N] + xws_ref[...]) + b2_ref[...]
    a = jnp.maximum(a, 0.0)
    h2 = _bn_cols(a, g2_ref[...], be2_ref[...])
    h = h_ref[...]

    wih1 = wih1_ref[...]
    g1 = (lax.dot_general(h, wih1[:, :D], (((1,), (1,)), ((), ())),
                          preferred_element_type=jnp.float32)
          + lax.dot_general(h2, wih1[:, D:], (((1,), (1,)), ((), ())),
                            preferred_element_type=jnp.float32)
          + bsum1_ref[...])
    i1 = jax.nn.sigmoid(g1[:, :D])
    gg1 = jnp.tanh(g1[:, 2 * D:3 * D])
    o1 = jax.nn.sigmoid(g1[:, 3 * D:])
    H1 = o1 * jnp.tanh(i1 * gg1)

    g2 = (lax.dot_general(H1, wih2_ref[...], (((1,), (1,)), ((), ())),
                          preferred_element_type=jnp.float32)
          + bsum2_ref[...])
    i2 = jax.nn.sigmoid(g2[:, :D])
    gg2 = jnp.tanh(g2[:, 2 * D:3 * D])
    o2 = jax.nn.sigmoid(g2[:, 3 * D:])
    H2 = o2 * jnp.tanh(i2 * gg2)

    wlin = wlin_ref[...]
    Hr1 = jnp.maximum(H1, 0.0)
    Hr2 = jnp.maximum(H2, 0.0)
    Hr3 = jnp.maximum(x_ref[...], 0.0)
    o = (lax.dot_general(Hr1, wlin[:, :D], (((1,), (1,)), ((), ())),
                         preferred_element_type=jnp.float32)
         + lax.dot_general(Hr2, wlin[:, D:2 * D], (((1,), (1,)), ((), ())),
                           preferred_element_type=jnp.float32)
         + lax.dot_general(Hr3, wlin[:, 2 * D:], (((1,), (1,)), ((), ())),
                           preferred_element_type=jnp.float32)
         + blin_ref[...])
    out_ref[...] = jnp.tanh(o)


def _tc_c(aggp, xws2, dis, b2, gamma2, beta2, h, x,
          Wih1, bsum1, Wih2, bsum2, Wlin, blin):
    return pl.pallas_call(
        _tc_c_body,
        out_shape=jax.ShapeDtypeStruct((N, 1), jnp.float32),
    )(aggp, xws2, dis, b2.reshape(1, D), gamma2.reshape(1, D),
      beta2.reshape(1, D), h, x, Wih1, bsum1.reshape(1, 4 * D),
      Wih2, bsum2.reshape(1, 4 * D), Wlin, blin.reshape(1, 1))


def kernel(x, edge_index, edge_weight, W1, b1, gamma1, beta1,
           W2, b2, gamma2, beta2, Wih1, Whh1, bih1, bhh1,
           Wih2, Whh2, bih2, bhh2, Wlin, blin):
    pad = E2 - E
    padidx = jnp.arange(pad, dtype=jnp.int32) % N
    row_p = jnp.concatenate([edge_index[0], padidx]).reshape(PADC, CK)
    col_p = jnp.concatenate([edge_index[1], padidx]).reshape(PADC, CK)
    ew_p = jnp.concatenate(
        [edge_weight, jnp.zeros((pad,), jnp.float32)]).reshape(PADC, CK)
    zeros1 = jnp.zeros((NP,), jnp.float32)
    zeros2 = jnp.zeros((NP, D), jnp.float32)

    degp = _sc_deg()(col_p, ew_p, zeros1)
    xws1, dis = _tc_a(degp.reshape(NC, NP, 1), x, W1)
    aggp1 = _sc_agg()(row_p, col_p, ew_p, xws1, zeros2)
    h, xws2 = _tc_b(aggp1, xws1, dis, b1, gamma1, beta1, W2)
    aggp2 = _sc_agg()(row_p, col_p, ew_p, xws2, zeros2)
    out = _tc_c(aggp2, xws2, dis, b2, gamma2, beta2, h, x,
                Wih1, bih1 + bhh1, Wih2, bih2 + bhh2, Wlin, blin)
    return out


# speedup vs baseline: 25.3431x; 1.0396x over previous
import functools

import jax
import jax.numpy as jnp
from jax import lax
from jax.experimental import pallas as pl
from jax.experimental.pallas import tpu as pltpu
from jax.experimental.pallas import tpu_sc as plsc

N = 10000
E = 320000
D = 128

NC = 2
NS = 16
NW = NC * NS
CK = 128
EC = E // CK
NCH = EC // NW
XTRA = EC - NW * NCH
ZR = 32
NP = 10240
RPT = NP // NS


def _tile_copy(src, dst, s):
    pltpu.sync_copy(src.at[pl.ds(s * RPT, RPT)], dst.at[pl.ds(s * RPT, RPT)])

@functools.cache
def _mesh():
    return plsc.VectorSubcoreMesh(core_axis_name="c", subcore_axis_name="s",
                                  num_cores=NC, num_subcores=NS)


def _sc_deg_body(ei_hbm, ew_hbm, out_hbm, col6, ew6, zb1, acc, sem_i, sem_s):
    c = lax.axis_index("c")
    s = lax.axis_index("s")
    wid = s * NC + c
    cb = wid * NCH + jnp.minimum(wid, XTRA)
    extra = wid < XTRA

    def zb_init(j, carry):
        zb1[pl.ds(j * 16, 16)] = jnp.zeros((16,), jnp.float32)
        return carry

    lax.fori_loop(0, RPT // 16, zb_init, 0)
    pltpu.sync_copy(zb1, acc.at[pl.ds(s * RPT, RPT)])
    plsc.subcore_barrier()

    def fetch(i, k):
        e0 = (cb + i) * CK
        return ((ei_hbm.at[1, pl.ds(e0, CK)], col6.at[k], sem_i),
                (ew_hbm.at[pl.ds(e0, CK)], ew6.at[k], sem_i))

    def sextet(p, carry):
        for k in range(6):
            for args in fetch(p * 6 + k, k):
                pltpu.async_copy(*args)
        for k in range(6):
            for args in fetch(p * 6 + k, k):
                pltpu.make_async_copy(*args).wait()
        for k in range(6):
            pltpu.async_copy(ew6.at[k], acc.at[col6.at[k]], sem_s, add=True)
        for k in range(6):
            pltpu.make_async_copy(ew6.at[k], acc.at[col6.at[k]],
                                  sem_s).wait()
        return carry

    lax.fori_loop(0, NCH // 6, sextet, 0)

    @pl.when(extra)
    def _():
        for args in fetch(NCH, 0):
            pltpu.sync_copy(args[0], args[1])
        pltpu.sync_copy(ew6.at[0], acc.at[col6.at[0]], add=True)

    plsc.subcore_barrier()
    _tile_copy(acc, out_hbm.at[c], s)


@functools.cache
def _sc_deg():
    return pl.kernel(
        _sc_deg_body,
        out_type=jax.ShapeDtypeStruct((NC, NP), jnp.float32),
        mesh=_mesh(),
        scratch_types=[
            pltpu.VMEM((6, CK), jnp.int32),
            pltpu.VMEM((6, CK), jnp.float32),
            pltpu.VMEM((RPT,), jnp.float32),
            pltpu.VMEM_SHARED((NP,), jnp.float32),
            pltpu.SemaphoreType.DMA,
            pltpu.SemaphoreType.DMA,
        ],
    )


def _sc_agg_body(ei_hbm, ew_hbm, xws_hbm, out_hbm,
                 col_v, row_v, ew_v, rows_v, zb, acc,
                 sem_i, sem_c, sem_g, sem_s):
    c = lax.axis_index("c")
    s = lax.axis_index("s")
    wid = s * NC + c
    cb = wid * NCH + jnp.minimum(wid, XTRA)
    extra = wid < XTRA
    def zb_init(r, carry):
        for j in range(D // 16):
            zb[r, pl.ds(j * 16, 16)] = jnp.zeros((16,), jnp.float32)
        return carry

    lax.fori_loop(0, ZR, zb_init, 0)
    for t in range(RPT // ZR):
        pltpu.async_copy(zb, acc.at[pl.ds(s * RPT + t * ZR, ZR)],
                         sem_g.at[0])
    for t in range(RPT // ZR):
        pltpu.make_async_copy(zb, acc.at[pl.ds(s * RPT + t * ZR, ZR)],
                              sem_g.at[0]).wait()

    def idx_start(i, b):
        e0 = (cb + i) * CK
        pltpu.async_copy(ei_hbm.at[0, pl.ds(e0, CK)], row_v.at[b],
                         sem_i.at[b])
        pltpu.async_copy(ew_hbm.at[pl.ds(e0, CK)], ew_v.at[b], sem_i.at[b])

    def idx_wait(i, b):
        e0 = (cb + i) * CK
        pltpu.make_async_copy(ei_hbm.at[0, pl.ds(e0, CK)], row_v.at[b],
                              sem_i.at[b]).wait()
        pltpu.make_async_copy(ew_hbm.at[pl.ds(e0, CK)], ew_v.at[b],
                              sem_i.at[b]).wait()

    def col_start(i, b):
        pltpu.async_copy(ei_hbm.at[1, pl.ds((cb + i) * CK, CK)], col_v.at[b],
                         sem_c.at[b])

    def col_wait(i, b):
        pltpu.make_async_copy(ei_hbm.at[1, pl.ds((cb + i) * CK, CK)],
                              col_v.at[b], sem_c.at[b]).wait()

    def gather_start(i, b):
        pltpu.async_copy(xws_hbm.at[row_v.at[b]], rows_v.at[b], sem_g.at[b])

    def gather_wait(i, b):
        pltpu.make_async_copy(xws_hbm.at[row_v.at[b]], rows_v.at[b],
                              sem_g.at[b]).wait()

    def scatter_start(i, b):
        pltpu.async_copy(rows_v.at[b], acc.at[col_v.at[b]], sem_s.at[b],
                         add=True)

    def scatter_wait(i, b):
        pltpu.make_async_copy(rows_v.at[b], acc.at[col_v.at[b]],
                              sem_s.at[b]).wait()

    idx_start(0, 0)
    idx_start(1, 1)
    col_start(0, 0)
    plsc.subcore_barrier()
    idx_wait(0, 0)
    gather_start(0, 0)

    def pair(p, carry):
        for k in range(2):
            i = p * 2 + k
            b = k
            nb = 1 - k

            @pl.when(i + 1 < NCH)
            def _():
                idx_wait(i + 1, nb)

                @pl.when(i >= 1)
                def _():
                    scatter_wait(i - 1, nb)
                col_start(i + 1, nb)
                gather_start(i + 1, nb)

            gather_wait(i, b)

            def group(g, gcarry):
                w16 = ew_v[b, pl.ds(g * 16, 16)]
                for l in range(16):
                    e = g * 16 + l
                    w = w16[l]
                    for j in range(D // 16):
                        sl = pl.ds(j * 16, 16)
                        rows_v[b, e, sl] = rows_v[b, e, sl] * w
                return gcarry

            lax.fori_loop(0, CK // 16, group, 0)
            col_wait(i, b)
            scatter_start(i, b)

            @pl.when(i + 2 < NCH)
            def _():
                idx_start(i + 2, b)
        return carry

    lax.fori_loop(0, NCH // 2, pair, 0)
    scatter_wait(NCH - 2, 0)
    scatter_wait(NCH - 1, 1)

    @pl.when(extra)
    def _():
        e0 = (cb + NCH) * CK
        pltpu.sync_copy(ei_hbm.at[0, pl.ds(e0, CK)], row_v.at[0])
        pltpu.sync_copy(ei_hbm.at[1, pl.ds(e0, CK)], col_v.at[0])
        pltpu.sync_copy(ew_hbm.at[pl.ds(e0, CK)], ew_v.at[0])
        pltpu.async_copy(xws_hbm.at[row_v.at[0]], rows_v.at[0],
                         sem_g.at[0]).wait()

        def egroup(g, gcarry):
            w16 = ew_v[0, pl.ds(g * 16, 16)]
            for l in range(16):
                e = g * 16 + l
                w = w16[l]
                for j in range(D // 16):
                    sl = pl.ds(j * 16, 16)
                    rows_v[0, e, sl] = rows_v[0, e, sl] * w
            return gcarry

        lax.fori_loop(0, CK // 16, egroup, 0)
        pltpu.sync_copy(rows_v.at[0], acc.at[col_v.at[0]], add=True)

    plsc.subcore_barrier()
    _tile_copy(acc, out_hbm.at[c], s)


@functools.cache
def _sc_agg():
    return pl.kernel(
        _sc_agg_body,
        out_type=jax.ShapeDtypeStruct((NC, NP, D), jnp.float32),
        mesh=_mesh(),
        scratch_types=[
            pltpu.VMEM((2, CK), jnp.int32),
            pltpu.VMEM((2, CK), jnp.int32),
            pltpu.VMEM((2, CK), jnp.float32),
            pltpu.VMEM((2, CK, D), jnp.float32),
            pltpu.VMEM((ZR, D), jnp.float32),
            pltpu.VMEM_SHARED((NP, D), jnp.float32),
            pltpu.SemaphoreType.DMA((2,)),
            pltpu.SemaphoreType.DMA((2,)),
            pltpu.SemaphoreType.DMA((2,)),
            pltpu.SemaphoreType.DMA((2,)),
        ],
    )


def _tc_a_body(degp_ref, x_ref, w1_ref, xws_ref, dis_ref):
    deg = degp_ref[0, :N] + degp_ref[1, :N] + 1.0
    dis = lax.rsqrt(deg)
    xw = lax.dot_general(x_ref[...], w1_ref[...],
                         (((1,), (1,)), ((), ())),
                         preferred_element_type=jnp.float32)
    xws_ref[...] = xw * dis
    dis_ref[...] = dis


def _tc_a(degp, x, W1):
    return pl.pallas_call(
        _tc_a_body,
        out_shape=(jax.ShapeDtypeStruct((N, D), jnp.float32),
                   jax.ShapeDtypeStruct((N, 1), jnp.float32)),
    )(degp, x, W1)


def _bn_cols(a, gamma, beta):
    mu = jnp.mean(a, axis=0, keepdims=True)
    d = a - mu
    var = jnp.mean(d * d, axis=0, keepdims=True)
    return d * lax.rsqrt(var + 1e-5) * gamma + beta


def _tc_b_body(aggp_ref, xws_ref, dis_ref, b1_ref, g1_ref, be1_ref, w2_ref,
               h_ref, xws2_ref):
    dis = dis_ref[...]
    a = dis * (aggp_ref[0, :N] + aggp_ref[1, :N] + xws_ref[...]) + b1_ref[...]
    a = jnp.maximum(a, 0.0)
    h = _bn_cols(a, g1_ref[...], be1_ref[...])
    h_ref[...] = h
    xw2 = lax.dot_general(h, w2_ref[...], (((1,), (1,)), ((), ())),
                          preferred_element_type=jnp.float32)
    xws2_ref[...] = xw2 * dis


def _tc_b(aggp, xws1, dis, b1, gamma1, beta1, W2):
    return pl.pallas_call(
        _tc_b_body,
        out_shape=(jax.ShapeDtypeStruct((N, D), jnp.float32),
                   jax.ShapeDtypeStruct((N, D), jnp.float32)),
    )(aggp, xws1, dis, b1.reshape(1, D), gamma1.reshape(1, D),
      beta1.reshape(1, D), W2)


def _tc_c_body(aggp_ref, xws_ref, dis_ref, b2_ref, g2_ref, be2_ref,
               h_ref, x_ref, wih1_ref, bsum1_ref, wih2_ref, bsum2_ref,
               wlin_ref, blin_ref, out_ref):
    dis = dis_ref[...]
    a = dis * (aggp_ref[0, :N] + aggp_ref[1, :N] + xws_ref[...]) + b2_ref[...]
    a = jnp.maximum(a, 0.0)
    h2 = _bn_cols(a, g2_ref[...], be2_ref[...])
    h = h_ref[...]

    wih1 = wih1_ref[...]
    g1 = (lax.dot_general(h, wih1[:, :D], (((1,), (1,)), ((), ())),
                          preferred_element_type=jnp.float32)
          + lax.dot_general(h2, wih1[:, D:], (((1,), (1,)), ((), ())),
                            preferred_element_type=jnp.float32)
          + bsum1_ref[...])
    i1 = jax.nn.sigmoid(g1[:, :D])
    gg1 = jnp.tanh(g1[:, 2 * D:3 * D])
    o1 = jax.nn.sigmoid(g1[:, 3 * D:])
    H1 = o1 * jnp.tanh(i1 * gg1)

    g2 = (lax.dot_general(H1, wih2_ref[...], (((1,), (1,)), ((), ())),
                          preferred_element_type=jnp.float32)
          + bsum2_ref[...])
    i2 = jax.nn.sigmoid(g2[:, :D])
    gg2 = jnp.tanh(g2[:, 2 * D:3 * D])
    o2 = jax.nn.sigmoid(g2[:, 3 * D:])
    H2 = o2 * jnp.tanh(i2 * gg2)

    wlin = wlin_ref[...]
    Hr1 = jnp.maximum(H1, 0.0)
    Hr2 = jnp.maximum(H2, 0.0)
    Hr3 = jnp.maximum(x_ref[...], 0.0)
    o = (lax.dot_general(Hr1, wlin[:, :D], (((1,), (1,)), ((), ())),
                         preferred_element_type=jnp.float32)
         + lax.dot_general(Hr2, wlin[:, D:2 * D], (((1,), (1,)), ((), ())),
                           preferred_element_type=jnp.float32)
         + lax.dot_general(Hr3, wlin[:, 2 * D:], (((1,), (1,)), ((), ())),
                           preferred_element_type=jnp.float32)
         + blin_ref[...])
    out_ref[...] = jnp.tanh(o)


def _tc_c(aggp, xws2, dis, b2, gamma2, beta2, h, x,
          Wih1, bsum1, Wih2, bsum2, Wlin, blin):
    return pl.pallas_call(
        _tc_c_body,
        out_shape=jax.ShapeDtypeStruct((N, 1), jnp.float32),
    )(aggp, xws2, dis, b2.reshape(1, D), gamma2.reshape(1, D),
      beta2.reshape(1, D), h, x, Wih1, bsum1.reshape(1, 4 * D),
      Wih2, bsum2.reshape(1, 4 * D), Wlin, blin.reshape(1, 1))


def kernel(x, edge_index, edge_weight, W1, b1, gamma1, beta1,
           W2, b2, gamma2, beta2, Wih1, Whh1, bih1, bhh1,
           Wih2, Whh2, bih2, bhh2, Wlin, blin):
    degp = _sc_deg()(edge_index, edge_weight)
    xws1, dis = _tc_a(degp.reshape(NC, NP, 1), x, W1)
    aggp1 = _sc_agg()(edge_index, edge_weight, xws1)
    h, xws2 = _tc_b(aggp1, xws1, dis, b1, gamma1, beta1, W2)
    aggp2 = _sc_agg()(edge_index, edge_weight, xws2)
    out = _tc_c(aggp2, xws2, dis, b2, gamma2, beta2, h, x,
                Wih1, bih1 + bhh1, Wih2, bih2 + bhh2, Wlin, blin)
    return out


# speedup vs baseline: 25.5328x; 1.0075x over previous
import functools

import jax
import jax.numpy as jnp
from jax import lax
from jax.experimental import pallas as pl
from jax.experimental.pallas import tpu as pltpu
from jax.experimental.pallas import tpu_sc as plsc

N = 10000
E = 320000
D = 128

NC = 2
NS = 16
NW = NC * NS
CK = 128
EC = E // CK
NCH = EC // NW
XTRA = EC - NW * NCH
ZR = 32
NP = 10240
RPT = NP // NS


def _tile_copy(src, dst, s):
    pltpu.sync_copy(src.at[pl.ds(s * RPT, RPT)], dst.at[pl.ds(s * RPT, RPT)])

@functools.cache
def _mesh():
    return plsc.VectorSubcoreMesh(core_axis_name="c", subcore_axis_name="s",
                                  num_cores=NC, num_subcores=NS)


def _sc_deg_body(ei_hbm, ew_hbm, out_hbm, col6, ew6, zb1, acc, sem_i, sem_s):
    c = lax.axis_index("c")
    s = lax.axis_index("s")
    wid = s * NC + c
    cb = wid * NCH + jnp.minimum(wid, XTRA)
    extra = wid < XTRA

    def zb_init(j, carry):
        zb1[pl.ds(j * 16, 16)] = jnp.zeros((16,), jnp.float32)
        return carry

    lax.fori_loop(0, RPT // 16, zb_init, 0)
    pltpu.sync_copy(zb1, acc.at[pl.ds(s * RPT, RPT)])
    plsc.subcore_barrier()

    def fetch(i, k):
        e0 = (cb + i) * CK
        return ((ei_hbm.at[1, pl.ds(e0, CK)], col6.at[k], sem_i),
                (ew_hbm.at[pl.ds(e0, CK)], ew6.at[k], sem_i))

    def sextet(p, carry):
        for k in range(6):
            for args in fetch(p * 6 + k, k):
                pltpu.async_copy(*args)
        for k in range(6):
            for args in fetch(p * 6 + k, k):
                pltpu.make_async_copy(*args).wait()
        for k in range(6):
            pltpu.async_copy(ew6.at[k], acc.at[col6.at[k]], sem_s, add=True)
        for k in range(6):
            pltpu.make_async_copy(ew6.at[k], acc.at[col6.at[k]],
                                  sem_s).wait()
        return carry

    lax.fori_loop(0, NCH // 6, sextet, 0)

    @pl.when(extra)
    def _():
        for args in fetch(NCH, 0):
            pltpu.sync_copy(args[0], args[1])
        pltpu.sync_copy(ew6.at[0], acc.at[col6.at[0]], add=True)

    plsc.subcore_barrier()
    _tile_copy(acc, out_hbm.at[c], s)


@functools.cache
def _sc_deg():
    return pl.kernel(
        _sc_deg_body,
        out_type=jax.ShapeDtypeStruct((NC, NP), jnp.float32),
        mesh=_mesh(),
        scratch_types=[
            pltpu.VMEM((6, CK), jnp.int32),
            pltpu.VMEM((6, CK), jnp.float32),
            pltpu.VMEM((RPT,), jnp.float32),
            pltpu.VMEM_SHARED((NP,), jnp.float32),
            pltpu.SemaphoreType.DMA,
            pltpu.SemaphoreType.DMA,
        ],
    )


def _sc_agg_body(ei_hbm, ew_hbm, xws_hbm, out_hbm,
                 col_v, row_v, ew_v, rows_v, zb, acc,
                 sem_i, sem_c, sem_g, sem_s):
    c = lax.axis_index("c")
    s = lax.axis_index("s")
    wid = s * NC + c
    cb = wid * NCH + jnp.minimum(wid, XTRA)
    extra = wid < XTRA
    def zb_init(r, carry):
        for j in range(D // 16):
            zb[r, pl.ds(j * 16, 16)] = jnp.zeros((16,), jnp.float32)
        return carry

    lax.fori_loop(0, ZR, zb_init, 0)
    for t in range(RPT // ZR):
        pltpu.async_copy(zb, acc.at[pl.ds(s * RPT + t * ZR, ZR)],
                         sem_g.at[0])
    for t in range(RPT // ZR):
        pltpu.make_async_copy(zb, acc.at[pl.ds(s * RPT + t * ZR, ZR)],
                              sem_g.at[0]).wait()

    def idx_start(i, b):
        e0 = (cb + i) * CK
        pltpu.async_copy(ei_hbm.at[0, pl.ds(e0, CK)], row_v.at[b],
                         sem_i.at[b])
        pltpu.async_copy(ew_hbm.at[pl.ds(e0, CK)], ew_v.at[b], sem_i.at[b])

    def idx_wait(i, b):
        e0 = (cb + i) * CK
        pltpu.make_async_copy(ei_hbm.at[0, pl.ds(e0, CK)], row_v.at[b],
                              sem_i.at[b]).wait()
        pltpu.make_async_copy(ew_hbm.at[pl.ds(e0, CK)], ew_v.at[b],
                              sem_i.at[b]).wait()

    def col_start(i, b):
        pltpu.async_copy(ei_hbm.at[1, pl.ds((cb + i) * CK, CK)], col_v.at[b],
                         sem_c.at[b])

    def col_wait(i, b):
        pltpu.make_async_copy(ei_hbm.at[1, pl.ds((cb + i) * CK, CK)],
                              col_v.at[b], sem_c.at[b]).wait()

    def gather_start(i, b):
        pltpu.async_copy(xws_hbm.at[row_v.at[b]], rows_v.at[b], sem_g.at[b],
                         priority=1)

    def gather_wait(i, b):
        pltpu.make_async_copy(xws_hbm.at[row_v.at[b]], rows_v.at[b],
                              sem_g.at[b]).wait()

    def scatter_start(i, b):
        pltpu.async_copy(rows_v.at[b], acc.at[col_v.at[b]], sem_s.at[b],
                         add=True)

    def scatter_wait(i, b):
        pltpu.make_async_copy(rows_v.at[b], acc.at[col_v.at[b]],
                              sem_s.at[b]).wait()

    idx_start(0, 0)
    idx_start(1, 1)
    col_start(0, 0)
    plsc.subcore_barrier()
    idx_wait(0, 0)
    gather_start(0, 0)

    def pair(p, carry):
        for k in range(2):
            i = p * 2 + k
            b = k
            nb = 1 - k

            @pl.when(i + 1 < NCH)
            def _():
                idx_wait(i + 1, nb)

                @pl.when(i >= 1)
                def _():
                    scatter_wait(i - 1, nb)
                col_start(i + 1, nb)
                gather_start(i + 1, nb)

            gather_wait(i, b)

            def group(g, gcarry):
                w16 = ew_v[b, pl.ds(g * 16, 16)]
                for l in range(16):
                    e = g * 16 + l
                    w = w16[l]
                    for j in range(D // 16):
                        sl = pl.ds(j * 16, 16)
                        rows_v[b, e, sl] = rows_v[b, e, sl] * w
                return gcarry

            lax.fori_loop(0, CK // 16, group, 0)
            col_wait(i, b)
            scatter_start(i, b)

            @pl.when(i + 2 < NCH)
            def _():
                idx_start(i + 2, b)
        return carry

    lax.fori_loop(0, NCH // 2, pair, 0)
    scatter_wait(NCH - 2, 0)
    scatter_wait(NCH - 1, 1)

    @pl.when(extra)
    def _():
        e0 = (cb + NCH) * CK
        pltpu.sync_copy(ei_hbm.at[0, pl.ds(e0, CK)], row_v.at[0])
        pltpu.sync_copy(ei_hbm.at[1, pl.ds(e0, CK)], col_v.at[0])
        pltpu.sync_copy(ew_hbm.at[pl.ds(e0, CK)], ew_v.at[0])
        pltpu.async_copy(xws_hbm.at[row_v.at[0]], rows_v.at[0],
                         sem_g.at[0]).wait()

        def egroup(g, gcarry):
            w16 = ew_v[0, pl.ds(g * 16, 16)]
            for l in range(16):
                e = g * 16 + l
                w = w16[l]
                for j in range(D // 16):
                    sl = pl.ds(j * 16, 16)
                    rows_v[0, e, sl] = rows_v[0, e, sl] * w
            return gcarry

        lax.fori_loop(0, CK // 16, egroup, 0)
        pltpu.sync_copy(rows_v.at[0], acc.at[col_v.at[0]], add=True)

    plsc.subcore_barrier()
    _tile_copy(acc, out_hbm.at[c], s)


@functools.cache
def _sc_agg():
    return pl.kernel(
        _sc_agg_body,
        out_type=jax.ShapeDtypeStruct((NC, NP, D), jnp.float32),
        mesh=_mesh(),
        scratch_types=[
            pltpu.VMEM((2, CK), jnp.int32),
            pltpu.VMEM((2, CK), jnp.int32),
            pltpu.VMEM((2, CK), jnp.float32),
            pltpu.VMEM((2, CK, D), jnp.float32),
            pltpu.VMEM((ZR, D), jnp.float32),
            pltpu.VMEM_SHARED((NP, D), jnp.float32),
            pltpu.SemaphoreType.DMA((2,)),
            pltpu.SemaphoreType.DMA((2,)),
            pltpu.SemaphoreType.DMA((2,)),
            pltpu.SemaphoreType.DMA((2,)),
        ],
    )


def _tc_a_body(degp_ref, x_ref, w1_ref, xws_ref, dis_ref):
    deg = degp_ref[:N, 0:1] + degp_ref[:N, 1:2] + 1.0
    dis = lax.rsqrt(deg)
    xw = lax.dot_general(x_ref[...], w1_ref[...],
                         (((1,), (1,)), ((), ())),
                         preferred_element_type=jnp.float32)
    xws_ref[...] = xw * dis
    dis_ref[...] = dis


def _tc_a(degp, x, W1):
    return pl.pallas_call(
        _tc_a_body,
        out_shape=(jax.ShapeDtypeStruct((N, D), jnp.float32),
                   jax.ShapeDtypeStruct((N, 1), jnp.float32)),
    )(degp, x, W1)


def _bn_cols(a, gamma, beta):
    mu = jnp.mean(a, axis=0, keepdims=True)
    d = a - mu
    var = jnp.mean(d * d, axis=0, keepdims=True)
    return d * lax.rsqrt(var + 1e-5) * gamma + beta


def _tc_b_body(aggp_ref, xws_ref, dis_ref, b1_ref, g1_ref, be1_ref, w2_ref,
               h_ref, xws2_ref):
    dis = dis_ref[...]
    a = dis * (aggp_ref[0, :N] + aggp_ref[1, :N] + xws_ref[...]) + b1_ref[...]
    a = jnp.maximum(a, 0.0)
    h = _bn_cols(a, g1_ref[...], be1_ref[...])
    h_ref[...] = h
    xw2 = lax.dot_general(h, w2_ref[...], (((1,), (1,)), ((), ())),
                          preferred_element_type=jnp.float32)
    xws2_ref[...] = xw2 * dis


def _tc_b(aggp, xws1, dis, b1, gamma1, beta1, W2):
    return pl.pallas_call(
        _tc_b_body,
        out_shape=(jax.ShapeDtypeStruct((N, D), jnp.float32),
                   jax.ShapeDtypeStruct((N, D), jnp.float32)),
    )(aggp, xws1, dis, b1.reshape(1, D), gamma1.reshape(1, D),
      beta1.reshape(1, D), W2)


def _tc_c_body(aggp_ref, xws_ref, dis_ref, b2_ref, g2_ref, be2_ref,
               h_ref, x_ref, wih1_ref, bsum1_ref, wih2_ref, bsum2_ref,
               wlin_ref, blin_ref, out_ref):
    dis = dis_ref[...]
    a = dis * (aggp_ref[0, :N] + aggp_ref[1, :N] + xws_ref[...]) + b2_ref[...]
    a = jnp.maximum(a, 0.0)
    h2 = _bn_cols(a, g2_ref[...], be2_ref[...])
    h = h_ref[...]

    wih1 = jnp.concatenate([wih1_ref[:D], wih1_ref[2 * D:]], axis=0)
    bs1 = jnp.concatenate([bsum1_ref[:, :D], bsum1_ref[:, 2 * D:]], axis=1)
    g1 = (lax.dot_general(h, wih1[:, :D], (((1,), (1,)), ((), ())),
                          preferred_element_type=jnp.float32)
          + lax.dot_general(h2, wih1[:, D:], (((1,), (1,)), ((), ())),
                            preferred_element_type=jnp.float32)
          + bs1)
    i1 = jax.nn.sigmoid(g1[:, :D])
    gg1 = jnp.tanh(g1[:, D:2 * D])
    o1 = jax.nn.sigmoid(g1[:, 2 * D:])
    H1 = o1 * jnp.tanh(i1 * gg1)

    wih2 = jnp.concatenate([wih2_ref[:D], wih2_ref[2 * D:]], axis=0)
    bs2 = jnp.concatenate([bsum2_ref[:, :D], bsum2_ref[:, 2 * D:]], axis=1)
    g2 = (lax.dot_general(H1, wih2, (((1,), (1,)), ((), ())),
                          preferred_element_type=jnp.float32)
          + bs2)
    i2 = jax.nn.sigmoid(g2[:, :D])
    gg2 = jnp.tanh(g2[:, D:2 * D])
    o2 = jax.nn.sigmoid(g2[:, 2 * D:])
    H2 = o2 * jnp.tanh(i2 * gg2)

    wlin = wlin_ref[...]
    Hr1 = jnp.maximum(H1, 0.0)
    Hr2 = jnp.maximum(H2, 0.0)
    Hr3 = jnp.maximum(x_ref[...], 0.0)
    o = (lax.dot_general(Hr1, wlin[:, :D], (((1,), (1,)), ((), ())),
                         preferred_element_type=jnp.float32)
         + lax.dot_general(Hr2, wlin[:, D:2 * D], (((1,), (1,)), ((), ())),
                           preferred_element_type=jnp.float32)
         + lax.dot_general(Hr3, wlin[:, 2 * D:], (((1,), (1,)), ((), ())),
                           preferred_element_type=jnp.float32)
         + blin_ref[...])
    out_ref[...] = jnp.tanh(o)


def _tc_c(aggp, xws2, dis, b2, gamma2, beta2, h, x,
          Wih1, bsum1, Wih2, bsum2, Wlin, blin):
    return pl.pallas_call(
        _tc_c_body,
        out_shape=jax.ShapeDtypeStruct((N, 1), jnp.float32),
    )(aggp, xws2, dis, b2.reshape(1, D), gamma2.reshape(1, D),
      beta2.reshape(1, D), h, x, Wih1, bsum1.reshape(1, 4 * D),
      Wih2, bsum2.reshape(1, 4 * D), Wlin, blin.reshape(1, 1))


def kernel(x, edge_index, edge_weight, W1, b1, gamma1, beta1,
           W2, b2, gamma2, beta2, Wih1, Whh1, bih1, bhh1,
           Wih2, Whh2, bih2, bhh2, Wlin, blin):
    degp = _sc_deg()(edge_index, edge_weight)
    xws1, dis = _tc_a(degp.T, x, W1)
    aggp1 = _sc_agg()(edge_index, edge_weight, xws1)
    h, xws2 = _tc_b(aggp1, xws1, dis, b1, gamma1, beta1, W2)
    aggp2 = _sc_agg()(edge_index, edge_weight, xws2)
    out = _tc_c(aggp2, xws2, dis, b2, gamma2, beta2, h, x,
                Wih1, bih1 + bhh1, Wih2, bih2 + bhh2, Wlin, blin)
    return out
